# baseline (speedup 1.0000x reference)
import functools

import jax
import jax.numpy as jnp
from jax.experimental import pallas as pl
from jax.experimental.pallas import tpu as pltpu

_NEG = -3.0e38
_TOPK = 7


def _topk_kernel(q_ref, k_ref, imp_ref, vout_ref, iout_ref, bscr, vscr, iscr,
                 *, blk, total_k, nblk):
    i = pl.program_id(0)
    Q = q_ref.shape[0]

    @pl.when(i == 0)
    def _init():
        vscr[...] = jnp.full((Q, 128), _NEG, jnp.float32)
        iscr[...] = jnp.zeros((Q, 128), jnp.int32)
        bscr[1] = jnp.full((Q, blk), _NEG, jnp.float32)

    j = jnp.minimum(i, nblk - 1)
    q = q_ref[...]
    ks = k_ref[...]
    imp = imp_ref[...]

    qn = q / (jnp.sqrt(jnp.sum(q * q, axis=1, keepdims=True)) + 1e-8)
    kn = ks / (jnp.sqrt(jnp.sum(ks * ks, axis=1, keepdims=True)) + 1e-8)
    sims = jax.lax.dot_general(qn, kn, (((1,), (1,)), ((), ())),
                               preferred_element_type=jnp.float32)
    colc = jax.lax.broadcasted_iota(jnp.int32, (Q, blk), 1) + j * blk
    bcur = jnp.where(colc < total_k, sims * (1.0 + 0.3 * imp), _NEG)
    bscr[i % 2] = bcur

    b = bscr[1 - i % 2]
    col = jax.lax.broadcasted_iota(jnp.int32, (Q, blk), 1) + (i - 1) * blk
    vals = vscr[...]
    idxs = iscr[...]
    lane = jax.lax.broadcasted_iota(jnp.int32, (Q, 128), 1)
    for _ in range(_TOPK):
        m = jnp.max(b, axis=1, keepdims=True)
        eq = b == m
        mi = jnp.min(jnp.where(eq, col, jnp.int32(2**31 - 1)), axis=1,
                     keepdims=True)
        b = jnp.where(eq, _NEG, b)
        vs = jnp.roll(vals, 1, axis=1)
        ishift = jnp.roll(idxs, 1, axis=1)
        ge = vals >= m
        ge_s = (vs >= m) | (lane == 0)
        vals = jnp.where(ge, vals, jnp.where(ge_s, jnp.broadcast_to(m, (Q, 128)), vs))
        idxs = jnp.where(ge, idxs, jnp.where(ge_s, jnp.broadcast_to(mi, (Q, 128)), ishift))
    vscr[...] = vals
    iscr[...] = idxs

    @pl.when(i == nblk)
    def _out():
        vout_ref[...] = vals[:, :_TOPK]
        iout_ref[...] = idxs[:, :_TOPK]


@functools.partial(jax.jit, static_argnames=("interpret",))
def _run(queries, keys, importance, interpret=False):
    Q, D = queries.shape
    K = keys.shape[0]
    blk = 2048
    nblk = pl.cdiv(K, blk)
    imp2 = importance.reshape(1, K)
    kern = functools.partial(_topk_kernel, blk=blk, total_k=K, nblk=nblk)
    clamp = nblk - 1
    vals, idxs = pl.pallas_call(
        kern,
        grid=(nblk + 1,),
        in_specs=[
            pl.BlockSpec((Q, D), lambda i: (0, 0)),
            pl.BlockSpec((blk, D), lambda i: (jnp.minimum(i, clamp), 0)),
            pl.BlockSpec((1, blk), lambda i: (0, jnp.minimum(i, clamp))),
        ],
        out_specs=[
            pl.BlockSpec((Q, _TOPK), lambda i: (0, 0)),
            pl.BlockSpec((Q, _TOPK), lambda i: (0, 0)),
        ],
        out_shape=[
            jax.ShapeDtypeStruct((Q, _TOPK), jnp.float32),
            jax.ShapeDtypeStruct((Q, _TOPK), jnp.int32),
        ],
        scratch_shapes=[
            pltpu.VMEM((2, Q, blk), jnp.float32),
            pltpu.VMEM((Q, 128), jnp.float32),
            pltpu.VMEM((Q, 128), jnp.int32),
        ],
        interpret=interpret,
    )(queries, keys, imp2)
    return vals, idxs


def kernel(queries, keys, importance, k):
    del k
    return _run(queries, keys, importance)


# speedup vs baseline: 1.4391x; 1.4391x over previous
import functools

import jax
import jax.numpy as jnp
from jax.experimental import pallas as pl
from jax.experimental.pallas import tpu as pltpu

_NEG = -3.0e38
_TOPK = 7
_IMAX = jnp.iinfo(jnp.int32).max


def _topk_kernel(q_ref, k_ref, imp_ref, vout_ref, iout_ref, vscr, iscr,
                 *, blk, total_k, nblk):
    i = pl.program_id(0)
    Q = q_ref.shape[0]

    @pl.when(i == 0)
    def _init():
        vscr[...] = jnp.full((Q, _TOPK * 128), _NEG, jnp.float32)
        iscr[...] = jnp.zeros((Q, _TOPK * 128), jnp.int32)

    q = q_ref[...]
    ks = k_ref[...]
    imp = imp_ref[...]

    qn = q / (jnp.sqrt(jnp.sum(q * q, axis=1, keepdims=True)) + 1e-8)
    kn = ks / (jnp.sqrt(jnp.sum(ks * ks, axis=1, keepdims=True)) + 1e-8)
    sims = jax.lax.dot_general(qn, kn, (((1,), (1,)), ((), ())),
                               preferred_element_type=jnp.float32)
    colc = jax.lax.broadcasted_iota(jnp.int32, (Q, blk), 1) + i * blk
    b = jnp.where(colc < total_k, sims * (1.0 + 0.3 * imp), _NEG)

    vals = [vscr[:, l * 128:(l + 1) * 128] for l in range(_TOPK)]
    idxs = [iscr[:, l * 128:(l + 1) * 128] for l in range(_TOPK)]
    lane = jax.lax.broadcasted_iota(jnp.int32, (Q, 128), 1)
    for c in range(blk // 128):
        x = b[:, c * 128:(c + 1) * 128]
        xi = lane + (i * blk + c * 128)
        for l in range(_TOPK):
            v_old = vals[l]
            i_old = idxs[l]
            keep = v_old >= x
            vals[l] = jnp.where(keep, v_old, x)
            idxs[l] = jnp.where(keep, i_old, xi)
            x = jnp.where(keep, x, v_old)
            xi = jnp.where(keep, xi, i_old)
    for l in range(_TOPK):
        vscr[:, l * 128:(l + 1) * 128] = vals[l]
        iscr[:, l * 128:(l + 1) * 128] = idxs[l]

    @pl.when(i == nblk - 1)
    def _out():
        vstk = list(vals)
        m7 = vstk[0]
        for l in range(1, _TOPK):
            m7 = jnp.maximum(m7, vstk[l])
        for j in range(_TOPK):
            m = jnp.max(m7, axis=1, keepdims=True)
            cand = jnp.full((Q, 128), _IMAX, jnp.int32)
            for l in range(_TOPK):
                cand = jnp.minimum(cand, jnp.where(vstk[l] == m, idxs[l], _IMAX))
            mi = jnp.min(cand, axis=1, keepdims=True)
            vout_ref[:, j:j + 1] = m
            iout_ref[:, j:j + 1] = mi
            m7 = _NEG
            for l in range(_TOPK):
                vstk[l] = jnp.where((vstk[l] == m) & (idxs[l] == mi), _NEG,
                                    vstk[l])
                m7 = jnp.maximum(m7, vstk[l])


@functools.partial(jax.jit, static_argnames=("interpret",))
def _run(queries, keys, importance, interpret=False):
    Q, D = queries.shape
    K = keys.shape[0]
    blk = 2048
    nblk = pl.cdiv(K, blk)
    imp2 = importance.reshape(1, K)
    kern = functools.partial(_topk_kernel, blk=blk, total_k=K, nblk=nblk)
    vals, idxs = pl.pallas_call(
        kern,
        grid=(nblk,),
        in_specs=[
            pl.BlockSpec((Q, D), lambda i: (0, 0)),
            pl.BlockSpec((blk, D), lambda i: (i, 0)),
            pl.BlockSpec((1, blk), lambda i: (0, i)),
        ],
        out_specs=[
            pl.BlockSpec((Q, _TOPK), lambda i: (0, 0)),
            pl.BlockSpec((Q, _TOPK), lambda i: (0, 0)),
        ],
        out_shape=[
            jax.ShapeDtypeStruct((Q, _TOPK), jnp.float32),
            jax.ShapeDtypeStruct((Q, _TOPK), jnp.int32),
        ],
        scratch_shapes=[
            pltpu.VMEM((Q, _TOPK * 128), jnp.float32),
            pltpu.VMEM((Q, _TOPK * 128), jnp.int32),
        ],
        interpret=interpret,
    )(queries, keys, imp2)
    return vals, idxs


def kernel(queries, keys, importance, k):
    del k
    return _run(queries, keys, importance)


# speedup vs baseline: 1.5198x; 1.0561x over previous
import functools

import jax
import jax.numpy as jnp
from jax.experimental import pallas as pl
from jax.experimental.pallas import tpu as pltpu

_NEG = -3.0e38
_TOPK = 7
_IMAX = jnp.iinfo(jnp.int32).max


def _topk_kernel(q_ref, k_ref, imp_ref, vout_ref, iout_ref, vscr, iscr,
                 *, blk, total_k, nblk):
    i = pl.program_id(0)
    Q = q_ref.shape[0]

    @pl.when(i == 0)
    def _init():
        vscr[...] = jnp.full((Q, _TOPK * 128), _NEG, jnp.float32)
        iscr[...] = jnp.zeros((Q, _TOPK * 128), jnp.int32)

    q = q_ref[...]
    ks = k_ref[...]
    imp = imp_ref[...]

    qn = q / (jnp.sqrt(jnp.sum(q * q, axis=1, keepdims=True)) + 1e-8)
    s2 = jnp.sum(ks * ks, axis=1, keepdims=True)
    rec = 1.0 / (s2 * jax.lax.rsqrt(s2) + 1e-8)
    kn = ks * rec
    sims = jax.lax.dot_general(qn, kn, (((1,), (1,)), ((), ())),
                               preferred_element_type=jnp.float32)
    colc = jax.lax.broadcasted_iota(jnp.int32, (Q, blk), 1) + i * blk
    b = jnp.where(colc < total_k, sims * (1.0 + 0.3 * imp), _NEG)

    vals = [vscr[:, l * 128:(l + 1) * 128] for l in range(_TOPK)]
    idxs = [iscr[:, l * 128:(l + 1) * 128] for l in range(_TOPK)]
    lane = jax.lax.broadcasted_iota(jnp.int32, (Q, 128), 1)
    for c in range(blk // 128):
        x = b[:, c * 128:(c + 1) * 128]
        xi = lane + (i * blk + c * 128)
        for l in range(_TOPK):
            v_old = vals[l]
            i_old = idxs[l]
            keep = v_old >= x
            vals[l] = jnp.where(keep, v_old, x)
            idxs[l] = jnp.where(keep, i_old, xi)
            x = jnp.where(keep, x, v_old)
            xi = jnp.where(keep, xi, i_old)
    for l in range(_TOPK):
        vscr[:, l * 128:(l + 1) * 128] = vals[l]
        iscr[:, l * 128:(l + 1) * 128] = idxs[l]

    @pl.when(i == nblk - 1)
    def _out():
        vstk = list(vals)
        m7 = vstk[0]
        for l in range(1, _TOPK):
            m7 = jnp.maximum(m7, vstk[l])
        for j in range(_TOPK):
            m = jnp.max(m7, axis=1, keepdims=True)
            cand = jnp.full((Q, 128), _IMAX, jnp.int32)
            for l in range(_TOPK):
                cand = jnp.minimum(cand, jnp.where(vstk[l] == m, idxs[l], _IMAX))
            mi = jnp.min(cand, axis=1, keepdims=True)
            vout_ref[:, j:j + 1] = m
            iout_ref[:, j:j + 1] = mi
            m7 = _NEG
            for l in range(_TOPK):
                vstk[l] = jnp.where((vstk[l] == m) & (idxs[l] == mi), _NEG,
                                    vstk[l])
                m7 = jnp.maximum(m7, vstk[l])


@functools.partial(jax.jit, static_argnames=("interpret",))
def _run(queries, keys, importance, interpret=False):
    Q, D = queries.shape
    K = keys.shape[0]
    blk = 2048
    nblk = pl.cdiv(K, blk)
    imp2 = importance.reshape(1, K)
    kern = functools.partial(_topk_kernel, blk=blk, total_k=K, nblk=nblk)
    vals, idxs = pl.pallas_call(
        kern,
        grid=(nblk,),
        in_specs=[
            pl.BlockSpec((Q, D), lambda i: (0, 0)),
            pl.BlockSpec((blk, D), lambda i: (i, 0)),
            pl.BlockSpec((1, blk), lambda i: (0, i)),
        ],
        out_specs=[
            pl.BlockSpec((Q, _TOPK), lambda i: (0, 0)),
            pl.BlockSpec((Q, _TOPK), lambda i: (0, 0)),
        ],
        out_shape=[
            jax.ShapeDtypeStruct((Q, _TOPK), jnp.float32),
            jax.ShapeDtypeStruct((Q, _TOPK), jnp.int32),
        ],
        scratch_shapes=[
            pltpu.VMEM((Q, _TOPK * 128), jnp.float32),
            pltpu.VMEM((Q, _TOPK * 128), jnp.int32),
        ],
        interpret=interpret,
    )(queries, keys, imp2)
    return vals, idxs


def kernel(queries, keys, importance, k):
    del k
    return _run(queries, keys, importance)


# speedup vs baseline: 2.0154x; 1.3261x over previous
import functools

import jax
import jax.numpy as jnp
from jax.experimental import pallas as pl
from jax.experimental.pallas import tpu as pltpu

_NEG = -3.0e38
_TOPK = 7
_IMAX = jnp.iinfo(jnp.int32).max


def _topk_kernel(q_ref, k_ref, imp_ref, vout_ref, iout_ref, vscr, iscr,
                 *, blk, total_k, nblk):
    i = pl.program_id(0)
    Q = q_ref.shape[0]

    @pl.when(i == 0)
    def _init():
        vscr[...] = jnp.full((Q, _TOPK * 128), _NEG, jnp.float32)
        iscr[...] = jnp.zeros((Q, _TOPK * 128), jnp.int32)

    q = q_ref[...]
    ks = k_ref[...]
    imp = imp_ref[...]

    qn = q / (jnp.sqrt(jnp.sum(q * q, axis=1, keepdims=True)) + 1e-8)
    s2 = jnp.sum(ks * ks, axis=1, keepdims=True)
    rec = 1.0 / (s2 * jax.lax.rsqrt(s2) + 1e-8)
    kn = ks * rec
    sims = jax.lax.dot_general(qn, kn, (((1,), (1,)), ((), ())),
                               preferred_element_type=jnp.float32)
    colc = jax.lax.broadcasted_iota(jnp.int32, (Q, blk), 1) + i * blk
    b = jnp.where(colc < total_k, sims * (1.0 + 0.3 * imp), _NEG)

    vals = [vscr[:, l * 128:(l + 1) * 128] for l in range(_TOPK)]
    idxs = [iscr[:, l * 128:(l + 1) * 128] for l in range(_TOPK)]
    lane = jax.lax.broadcasted_iota(jnp.int32, (Q, 128), 1)
    for c in range(blk // 128):
        x = b[:, c * 128:(c + 1) * 128]
        xi = lane + (i * blk + c * 128)
        for l in range(_TOPK):
            v_old = vals[l]
            i_old = idxs[l]
            keep = v_old >= x
            vals[l] = jnp.where(keep, v_old, x)
            idxs[l] = jnp.where(keep, i_old, xi)
            x = jnp.where(keep, x, v_old)
            xi = jnp.where(keep, xi, i_old)
    for l in range(_TOPK):
        vscr[:, l * 128:(l + 1) * 128] = vals[l]
        iscr[:, l * 128:(l + 1) * 128] = idxs[l]

    @pl.when(i == nblk - 1)
    def _out():
        vstk = list(vals)
        m7 = vstk[0]
        for l in range(1, _TOPK):
            m7 = jnp.maximum(m7, vstk[l])
        for j in range(_TOPK):
            m = jnp.max(m7, axis=1, keepdims=True)
            cand = jnp.full((Q, 128), _IMAX, jnp.int32)
            for l in range(_TOPK):
                cand = jnp.minimum(cand, jnp.where(vstk[l] == m, idxs[l], _IMAX))
            mi = jnp.min(cand, axis=1, keepdims=True)
            vout_ref[:, j:j + 1] = m
            iout_ref[:, j:j + 1] = mi
            m7 = _NEG
            for l in range(_TOPK):
                vstk[l] = jnp.where((vstk[l] == m) & (idxs[l] == mi), _NEG,
                                    vstk[l])
                m7 = jnp.maximum(m7, vstk[l])


@functools.partial(jax.jit, static_argnames=("interpret",))
def _run(queries, keys, importance, interpret=False):
    Q, D = queries.shape
    K = keys.shape[0]
    blk = 4096
    nblk = pl.cdiv(K, blk)
    imp2 = importance.reshape(1, K)
    kern = functools.partial(_topk_kernel, blk=blk, total_k=K, nblk=nblk)
    vals, idxs = pl.pallas_call(
        kern,
        grid=(nblk,),
        in_specs=[
            pl.BlockSpec((Q, D), lambda i: (0, 0)),
            pl.BlockSpec((blk, D), lambda i: (i, 0)),
            pl.BlockSpec((1, blk), lambda i: (0, i)),
        ],
        out_specs=[
            pl.BlockSpec((Q, _TOPK), lambda i: (0, 0)),
            pl.BlockSpec((Q, _TOPK), lambda i: (0, 0)),
        ],
        out_shape=[
            jax.ShapeDtypeStruct((Q, _TOPK), jnp.float32),
            jax.ShapeDtypeStruct((Q, _TOPK), jnp.int32),
        ],
        scratch_shapes=[
            pltpu.VMEM((Q, _TOPK * 128), jnp.float32),
            pltpu.VMEM((Q, _TOPK * 128), jnp.int32),
        ],
        interpret=interpret,
    )(queries, keys, imp2)
    return vals, idxs


def kernel(queries, keys, importance, k):
    del k
    return _run(queries, keys, importance)


# speedup vs baseline: 2.3075x; 1.1449x over previous
import functools

import jax
import jax.numpy as jnp
from jax.experimental import pallas as pl
from jax.experimental.pallas import tpu as pltpu

_NEG = -3.0e38
_TOPK = 7
_IMAX = jnp.iinfo(jnp.int32).max


def _topk_kernel(q_ref, k_ref, imp_ref, vout_ref, iout_ref, vscr, iscr,
                 *, blk, total_k, nblk):
    i = pl.program_id(0)
    Q = q_ref.shape[0]

    @pl.when(i == 0)
    def _init():
        vscr[...] = jnp.full((Q, _TOPK * 128), _NEG, jnp.float32)
        iscr[...] = jnp.zeros((Q, _TOPK * 128), jnp.int32)

    q = q_ref[...]
    ks = k_ref[...]
    imp = imp_ref[...]

    qn = q / (jnp.sqrt(jnp.sum(q * q, axis=1, keepdims=True)) + 1e-8)
    s2 = jnp.sum(ks * ks, axis=1, keepdims=True)
    rec = 1.0 / (s2 * jax.lax.rsqrt(s2) + 1e-8)
    kn = ks * rec
    sims = jax.lax.dot_general(qn, kn, (((1,), (1,)), ((), ())),
                               preferred_element_type=jnp.float32)
    colc = jax.lax.broadcasted_iota(jnp.int32, (Q, blk), 1) + i * blk
    b = jnp.where(colc < total_k, sims * (1.0 + 0.3 * imp), _NEG)

    vals = [vscr[:, l * 128:(l + 1) * 128] for l in range(_TOPK)]
    idxs = [iscr[:, l * 128:(l + 1) * 128] for l in range(_TOPK)]
    lane = jax.lax.broadcasted_iota(jnp.int32, (Q, 128), 1)
    for c in range(blk // 128):
        x = b[:, c * 128:(c + 1) * 128]
        xi = lane + (i * blk + c * 128)
        for l in range(_TOPK):
            v_old = vals[l]
            i_old = idxs[l]
            keep = v_old >= x
            vals[l] = jnp.where(keep, v_old, x)
            idxs[l] = jnp.where(keep, i_old, xi)
            x = jnp.where(keep, x, v_old)
            xi = jnp.where(keep, xi, i_old)
    for l in range(_TOPK):
        vscr[:, l * 128:(l + 1) * 128] = vals[l]
        iscr[:, l * 128:(l + 1) * 128] = idxs[l]

    @pl.when(i == nblk - 1)
    def _out():
        vstk = list(vals)
        m7 = vstk[0]
        for l in range(1, _TOPK):
            m7 = jnp.maximum(m7, vstk[l])
        for j in range(_TOPK):
            m = jnp.max(m7, axis=1, keepdims=True)
            cand = jnp.full((Q, 128), _IMAX, jnp.int32)
            for l in range(_TOPK):
                cand = jnp.minimum(cand, jnp.where(vstk[l] == m, idxs[l], _IMAX))
            mi = jnp.min(cand, axis=1, keepdims=True)
            vout_ref[:, j:j + 1] = m
            iout_ref[:, j:j + 1] = mi
            m7 = _NEG
            for l in range(_TOPK):
                vstk[l] = jnp.where((vstk[l] == m) & (idxs[l] == mi), _NEG,
                                    vstk[l])
                m7 = jnp.maximum(m7, vstk[l])


@functools.partial(jax.jit, static_argnames=("interpret",))
def _run(queries, keys, importance, interpret=False):
    Q, D = queries.shape
    K = keys.shape[0]
    blk = 8192
    nblk = pl.cdiv(K, blk)
    imp2 = importance.reshape(1, K)
    kern = functools.partial(_topk_kernel, blk=blk, total_k=K, nblk=nblk)
    vals, idxs = pl.pallas_call(
        kern,
        grid=(nblk,),
        in_specs=[
            pl.BlockSpec((Q, D), lambda i: (0, 0)),
            pl.BlockSpec((blk, D), lambda i: (i, 0)),
            pl.BlockSpec((1, blk), lambda i: (0, i)),
        ],
        out_specs=[
            pl.BlockSpec((Q, _TOPK), lambda i: (0, 0)),
            pl.BlockSpec((Q, _TOPK), lambda i: (0, 0)),
        ],
        out_shape=[
            jax.ShapeDtypeStruct((Q, _TOPK), jnp.float32),
            jax.ShapeDtypeStruct((Q, _TOPK), jnp.int32),
        ],
        scratch_shapes=[
            pltpu.VMEM((Q, _TOPK * 128), jnp.float32),
            pltpu.VMEM((Q, _TOPK * 128), jnp.int32),
        ],
        interpret=interpret,
    )(queries, keys, imp2)
    return vals, idxs


def kernel(queries, keys, importance, k):
    del k
    return _run(queries, keys, importance)


# speedup vs baseline: 2.5233x; 1.0935x over previous
import functools

import jax
import jax.numpy as jnp
from jax.experimental import pallas as pl
from jax.experimental.pallas import tpu as pltpu

_NEG = -3.0e38
_TOPK = 7
_IMAX = jnp.iinfo(jnp.int32).max


def _topk_kernel(q_ref, k_ref, imp_ref, vout_ref, iout_ref, vscr, iscr,
                 *, blk, total_k, nblk):
    i = pl.program_id(0)
    Q = q_ref.shape[0]

    @pl.when(i == 0)
    def _init():
        vscr[...] = jnp.full((Q, _TOPK * 128), _NEG, jnp.float32)
        iscr[...] = jnp.zeros((Q, _TOPK * 128), jnp.int32)

    q = q_ref[...]
    ks = k_ref[...]
    imp = imp_ref[...]

    qn = q / (jnp.sqrt(jnp.sum(q * q, axis=1, keepdims=True)) + 1e-8)
    s2 = jnp.sum(ks * ks, axis=1, keepdims=True)
    rec = 1.0 / (s2 * jax.lax.rsqrt(s2) + 1e-8)
    kn = ks * rec
    sims = jax.lax.dot_general(qn, kn, (((1,), (1,)), ((), ())),
                               preferred_element_type=jnp.float32)
    colc = jax.lax.broadcasted_iota(jnp.int32, (Q, blk), 1) + i * blk
    b = jnp.where(colc < total_k, sims * (1.0 + 0.3 * imp), _NEG)

    vals = [vscr[:, l * 128:(l + 1) * 128] for l in range(_TOPK)]
    idxs = [iscr[:, l * 128:(l + 1) * 128] for l in range(_TOPK)]
    lane = jax.lax.broadcasted_iota(jnp.int32, (Q, 128), 1)
    for c in range(blk // 128):
        x = b[:, c * 128:(c + 1) * 128]
        xi = lane + (i * blk + c * 128)
        for l in range(_TOPK):
            v_old = vals[l]
            i_old = idxs[l]
            keep = v_old >= x
            vals[l] = jnp.where(keep, v_old, x)
            idxs[l] = jnp.where(keep, i_old, xi)
            x = jnp.where(keep, x, v_old)
            xi = jnp.where(keep, xi, i_old)
    for l in range(_TOPK):
        vscr[:, l * 128:(l + 1) * 128] = vals[l]
        iscr[:, l * 128:(l + 1) * 128] = idxs[l]

    @pl.when(i == nblk - 1)
    def _out():
        vstk = list(vals)
        m7 = vstk[0]
        for l in range(1, _TOPK):
            m7 = jnp.maximum(m7, vstk[l])
        for j in range(_TOPK):
            m = jnp.max(m7, axis=1, keepdims=True)
            cand = jnp.full((Q, 128), _IMAX, jnp.int32)
            for l in range(_TOPK):
                cand = jnp.minimum(cand, jnp.where(vstk[l] == m, idxs[l], _IMAX))
            mi = jnp.min(cand, axis=1, keepdims=True)
            vout_ref[:, j:j + 1] = m
            iout_ref[:, j:j + 1] = mi
            m7 = _NEG
            for l in range(_TOPK):
                vstk[l] = jnp.where((vstk[l] == m) & (idxs[l] == mi), _NEG,
                                    vstk[l])
                m7 = jnp.maximum(m7, vstk[l])


@functools.partial(jax.jit, static_argnames=("interpret",))
def _run(queries, keys, importance, interpret=False):
    Q, D = queries.shape
    K = keys.shape[0]
    blk = 12544
    nblk = pl.cdiv(K, blk)
    imp2 = importance.reshape(1, K)
    kern = functools.partial(_topk_kernel, blk=blk, total_k=K, nblk=nblk)
    vals, idxs = pl.pallas_call(
        kern,
        grid=(nblk,),
        in_specs=[
            pl.BlockSpec((Q, D), lambda i: (0, 0)),
            pl.BlockSpec((blk, D), lambda i: (i, 0)),
            pl.BlockSpec((1, blk), lambda i: (0, i)),
        ],
        out_specs=[
            pl.BlockSpec((Q, _TOPK), lambda i: (0, 0)),
            pl.BlockSpec((Q, _TOPK), lambda i: (0, 0)),
        ],
        out_shape=[
            jax.ShapeDtypeStruct((Q, _TOPK), jnp.float32),
            jax.ShapeDtypeStruct((Q, _TOPK), jnp.int32),
        ],
        scratch_shapes=[
            pltpu.VMEM((Q, _TOPK * 128), jnp.float32),
            pltpu.VMEM((Q, _TOPK * 128), jnp.int32),
        ],
        interpret=interpret,
    )(queries, keys, imp2)
    return vals, idxs


def kernel(queries, keys, importance, k):
    del k
    return _run(queries, keys, importance)
